# Initial kernel scaffold; baseline (speedup 1.0000x reference)
import jax
import jax.numpy as jnp
from jax.experimental import pallas as pl


def kernel(species, ecfp, bond_order, edge_src, edge_dst, distances, switch, angles, ang_distances, central_atom, angle_src, angle_dst, ang_switch, ang_edge_dst):
    raise NotImplementedError("write your pallas kernel here")


# speedup vs baseline: 1.7169x; 1.7169x over previous
import numpy as np
import jax
import jax.numpy as jnp
from jax.experimental import pallas as pl

_RADIAL_ETA = 16.0
_ANGULAR_ETA = 8.0
_RADIAL_DIV = 16
_ANGULAR_DIV = 4
_ZETA = 32.0
_ANGLE_SECTIONS = 4
_RADIAL_START = 0.8
_ANGULAR_START = 0.8
_CUTOFF_RAD = 5.2
_CUTOFF_ANG = 3.5

_SHIFT_R = np.linspace(_RADIAL_START, _CUTOFF_RAD, _RADIAL_DIV + 1)[:-1].astype(np.float32)
_SHIFT_A = np.linspace(_ANGULAR_START, _CUTOFF_ANG, _ANGULAR_DIV + 1)[:-1].astype(np.float32)
_SHIFT_Z = (np.linspace(0, np.pi, _ANGLE_SECTIONS + 1) + np.pi / (2 * _ANGLE_SECTIONS))[:-1].astype(np.float32)


def _radial_body(d_ref, sw_ref, c_ref, out_ref):
    d = d_ref[...]
    sw = sw_ref[...]
    c = c_ref[...]
    step = (_CUTOFF_RAD - _RADIAL_START) / _RADIAL_DIV
    lane = jax.lax.broadcasted_iota(jnp.int32, (1, 32), 1)
    shift = _RADIAL_START + step * (lane // 2).astype(jnp.float32)
    par = (lane % 2).astype(jnp.float32)
    t = 0.25 * jnp.exp(-_RADIAL_ETA * (d - shift) ** 2) * sw
    out_ref[...] = t * jnp.where(par == c, 1.0, 0.0)


def _angular_body(ang_ref, d12_ref, sc_ref, out_ref):
    ang = ang_ref[...]
    d12 = d12_ref[...]
    scale = sc_ref[...]
    lane = jax.lax.broadcasted_iota(jnp.int32, (1, 16), 1)
    a = (lane // 4).astype(jnp.float32)
    z = (lane % 4).astype(jnp.float32)
    sz = np.float32(np.pi / (2 * _ANGLE_SECTIONS)) + np.float32(np.pi / _ANGLE_SECTIONS) * z
    sa = _ANGULAR_START + ((_CUTOFF_ANG - _ANGULAR_START) / _ANGULAR_DIV) * a
    f1 = (0.5 + 0.5 * jnp.cos(ang - sz)) ** _ZETA
    f2 = jnp.exp(-_ANGULAR_ETA * (d12 - sa) ** 2)
    out_ref[...] = f1 * f2 * scale


def _compute_radial_terms(d, sw, c):
    e = d.shape[0]
    blk = 1000 if e % 1000 == 0 else 8
    assert e % blk == 0
    grid = (e // blk,)
    f = pl.pallas_call(
        _radial_body,
        grid=grid,
        in_specs=[pl.BlockSpec((blk, 1), lambda i: (i, 0))] * 3,
        out_specs=pl.BlockSpec((blk, 32), lambda i: (i, 0)),
        out_shape=jax.ShapeDtypeStruct((e, 32), jnp.float32),
    )
    return f(d[:, None], sw[:, None], c[:, None])


def _compute_angular_terms(ang, d12, scale):
    t = ang.shape[0]
    blk = 1000 if t % 1000 == 0 else 8
    assert t % blk == 0
    grid = (t // blk,)
    f = pl.pallas_call(
        _angular_body,
        grid=grid,
        in_specs=[pl.BlockSpec((blk, 1), lambda i: (i, 0))] * 3,
        out_specs=pl.BlockSpec((blk, 16), lambda i: (i, 0)),
        out_shape=jax.ShapeDtypeStruct((t, 16), jnp.float32),
    )
    return f(ang[:, None], d12[:, None], scale[:, None])


def kernel(species, ecfp, bond_order, edge_src, edge_dst, distances, switch,
           angles, ang_distances, central_atom, angle_src, angle_dst,
           ang_switch, ang_edge_dst):
    n_atoms = species.shape[0]
    num_species = 4
    num_pair = num_species * (num_species + 1) // 2
    idx = ((species == 6).astype(jnp.int32)
           + 2 * (species == 7).astype(jnp.int32)
           + 3 * (species == 8).astype(jnp.int32))
    c = jnp.logical_or(bond_order == 3, bond_order == 5).astype(jnp.float32)
    rad_terms = _compute_radial_terms(distances, switch, c)
    radial_index = edge_src * num_species + idx[edge_dst]
    radial_aev = jax.ops.segment_sum(rad_terms, radial_index, num_species * n_atoms)
    radial_aev = radial_aev.reshape(n_atoms, num_species * 32)

    d12 = 0.5 * (ang_distances[angle_src] + ang_distances[angle_dst])
    scale = 2.0 * ang_switch[angle_src] * ang_switch[angle_dst]
    ang_terms = _compute_angular_terms(angles, d12, scale)
    s1, s2 = np.triu_indices(num_species, 0)
    triu = np.zeros((num_species, num_species), dtype=np.int32)
    triu[s1, s2] = np.arange(s1.shape[0], dtype=np.int32)
    triu[s2, s1] = triu[s1, s2]
    triu = jnp.asarray(triu)
    spec_d = idx[ang_edge_dst]
    pair = triu[spec_d[angle_src], spec_d[angle_dst]]
    angular_index = central_atom * num_pair + pair
    angular_aev = jax.ops.segment_sum(ang_terms, angular_index, num_pair * n_atoms)
    angular_aev = angular_aev.reshape(n_atoms, num_pair * 16)
    return jnp.concatenate((ecfp, radial_aev, angular_aev), axis=-1)


# speedup vs baseline: 66.1296x; 38.5168x over previous
import functools
import numpy as np
import jax
import jax.numpy as jnp
from jax import lax
from jax.experimental import pallas as pl
from jax.experimental.pallas import tpu as pltpu
from jax.experimental.pallas import tpu_sc as plsc

NC, NS, L = 2, 16, 16
NW = NC * NS

N_ATOMS = 50000
E_RAD = 800000
E_ANG = 400000
N_ANG = 1400000
NB = 4
APB = N_ATOMS // NB

C = 1536
GROUPS = C // 16
CH_RAD = -(-E_RAD // C)
CH_TRI = -(-N_ANG // C)
CH_TAB = -(-E_ANG // C)
CAP_RAD = 26624
CAP_ANG = 44544
F = 128

R_RAD = 100000
RA_RAD = 100016
GARB_RAD = 100000
R_ANG = 125000
RA_ANG = 125008
GARB_ANG = 125000
BROWS = 125

_SINP = (9.9999924135e-01, -1.6665679619e-01, 8.3132250799e-03, -1.8523448330e-04)
_COSP = (9.9999995325e-01, -4.9999905063e-01, 4.1663578931e-02, -1.3853666933e-03,
         2.3153174156e-05)
_HALF_PI = float(np.pi / 2)
_SHIFT_Z = [float(np.pi / 8 + k * np.pi / 4) for k in range(4)]
_CZ = [float(np.cos(z)) for z in _SHIFT_Z]
_SZ = [float(np.sin(z)) for z in _SHIFT_Z]
_SHIFT_A = [0.8 + 0.675 * a for a in range(4)]
_SHIFT_R = [0.8 + 0.275 * j for j in range(16)]

_MESH = plsc.VectorSubcoreMesh(core_axis_name="c", subcore_axis_name="s",
                               num_cores=NC, num_subcores=NS)


def _lane():
    return lax.iota(jnp.int32, 16)


def _splat(x, dtype=jnp.int32):
    return jnp.full((16,), x, dtype=dtype)


def _wid():
    return lax.axis_index("s") * NC + lax.axis_index("c")


def _extract(vec16, pos):
    return jnp.sum(jnp.where(_lane() == pos, vec16, 0))


def _spec_idx(s):
    return ((s == 6).astype(jnp.int32) + 2 * (s == 7).astype(jnp.int32)
            + 3 * (s == 8).astype(jnp.int32))


def _append(stags, fields, mask, cnt, kb, outs, base):
    n = jnp.sum(mask.astype(jnp.int32))
    for sref, f in zip(stags, fields):
        plsc.store_compressed(sref.at[pl.ds(cnt, 16)], f, mask=mask)
    new = cnt + n
    do = new >= F

    @pl.when(do)
    def _():
        for sref, oref in zip(stags, outs):
            pltpu.sync_copy(sref.at[pl.ds(0, F)], oref.at[pl.ds(base + kb * F, F)])
            sref[pl.ds(0, 16)] = sref[pl.ds(F, 16)]

    doi = do.astype(jnp.int32)
    return new - F * doi, kb + doi


def _drain(stags, cnt, kb, outs, base):
    @pl.when(cnt > 0)
    def _():
        for sref, oref in zip(stags, outs):
            pltpu.sync_copy(sref.at[pl.ds(0, F)], oref.at[pl.ds(base + kb * F, F)])
    return kb * F + cnt


def _counts_vec(totals):
    v = _splat(0)
    for b, t in enumerate(totals):
        v = jnp.where(_lane() == b, _splat(t), v)
    return v


def _k1_body(angd, angsw, angdst, species, tab, dv, swv, dstv, spv, stag, sem):
    w = _wid()
    nk = (CH_TAB - w + NW - 1) // NW

    def chunk(k, carry):
        c = w + k * NW
        off = jnp.minimum(c * C, E_ANG - C)
        pltpu.sync_copy(angd.at[pl.ds(off, C)], dv)
        pltpu.sync_copy(angsw.at[pl.ds(off, C)], swv)
        pltpu.sync_copy(angdst.at[pl.ds(off, C)], dstv)
        for j in range(C // 128):
            pltpu.async_copy(species.at[dstv.at[pl.ds(j * 128, 128)]],
                             spv.at[pl.ds(j * 128, 128)], sem)
        for j in range(C // 128):
            pltpu.make_async_copy(species.at[dstv.at[pl.ds(j * 128, 128)]],
                                  spv.at[pl.ds(j * 128, 128)], sem).wait()

        def grp(g, _):
            row = g * 16 + _lane()
            d = plsc.load_gather(dv, [row])
            sw = plsc.load_gather(swv, [row])
            sp = plsc.load_gather(spv, [row])
            spf = _spec_idx(sp).astype(jnp.float32)
            plsc.store_scatter(stag, [row, _splat(0)], d)
            plsc.store_scatter(stag, [row, _splat(1)], sw)
            plsc.store_scatter(stag, [row, _splat(2)], spf)
            return 0

        lax.fori_loop(0, GROUPS, grp, 0)
        pltpu.sync_copy(stag, tab.at[pl.ds(off, C)])
        return 0

    lax.fori_loop(0, nk, chunk, 0)


def _k1(angd, angsw, angdst, species):
    f = pl.kernel(
        _k1_body,
        out_type=jax.ShapeDtypeStruct((E_ANG, 16), jnp.float32),
        mesh=_MESH,
        compiler_params=pltpu.CompilerParams(needs_layout_passes=False, use_tc_tiling_on_sc=False),
        scratch_types=[
            pltpu.VMEM((C,), jnp.float32),
            pltpu.VMEM((C,), jnp.float32),
            pltpu.VMEM((C,), jnp.int32),
            pltpu.VMEM((C,), jnp.int32),
            pltpu.VMEM((C, 16), jnp.float32),
            pltpu.SemaphoreType.DMA,
        ],
    )
    return f(angd, angsw, angdst, species)


def _k2_body(bo_h, src_h, dst_h, d_h, sw_h, species,
             rd_h, rsw_h, rrow_h, cnts_h,
             bov, srcv, dstv, dv, swv, spv, sd0, sd1, sd2, sd3,
             ss0, ss1, ss2, ss3, sr0, sr1, sr2, sr3, sem):
    w = _wid()
    nk = (CH_RAD - w + NW - 1) // NW
    sd = [sd0, sd1, sd2, sd3]
    ss = [ss0, ss1, ss2, ss3]
    sr = [sr0, sr1, sr2, sr3]

    def chunk(k, carry):
        c = w + k * NW
        off = jnp.minimum(c * C, E_RAD - C)
        lo = c * C
        pltpu.sync_copy(bo_h.at[pl.ds(off, C)], bov)
        pltpu.sync_copy(src_h.at[pl.ds(off, C)], srcv)
        pltpu.sync_copy(dst_h.at[pl.ds(off, C)], dstv)
        pltpu.sync_copy(d_h.at[pl.ds(off, C)], dv)
        pltpu.sync_copy(sw_h.at[pl.ds(off, C)], swv)
        for j in range(C // 128):
            pltpu.async_copy(species.at[dstv.at[pl.ds(j * 128, 128)]],
                             spv.at[pl.ds(j * 128, 128)], sem)
        for j in range(C // 128):
            pltpu.make_async_copy(species.at[dstv.at[pl.ds(j * 128, 128)]],
                                  spv.at[pl.ds(j * 128, 128)], sem).wait()

        def grp(g, cr):
            row = g * 16 + _lane()
            gid = off + row
            valid = gid >= lo
            bo = plsc.load_gather(bov, [row])
            src = plsc.load_gather(srcv, [row])
            d = plsc.load_gather(dv, [row])
            sw = plsc.load_gather(swv, [row])
            sp = _spec_idx(plsc.load_gather(spv, [row]))
            ch = (jnp.logical_or(bo == 3, bo == 5)).astype(jnp.int32)
            bin_ = ((src >= APB).astype(jnp.int32)
                    + (src >= 2 * APB).astype(jnp.int32)
                    + (src >= 3 * APB).astype(jnp.int32))
            rrow = (src - bin_ * APB) * 8 + sp * 2 + ch
            cs = list(cr)
            for b in range(NB):
                m = jnp.logical_and(bin_ == b, valid)
                base = (w * NB + b) * CAP_RAD
                cs[b], cs[NB + b] = _append(
                    [sd[b], ss[b], sr[b]], [d, sw, rrow], m,
                    cs[b], cs[NB + b], [rd_h, rsw_h, rrow_h], base)
            return tuple(cs)

        return lax.fori_loop(0, GROUPS, grp, carry)

    carry = lax.fori_loop(0, nk, chunk, (jnp.int32(0),) * (2 * NB))
    totals = [_drain([sd[b], ss[b], sr[b]], carry[b], carry[NB + b],
                     [rd_h, rsw_h, rrow_h], (w * NB + b) * CAP_RAD)
              for b in range(NB)]
    sr0[pl.ds(0, 16)] = _counts_vec(totals)
    pltpu.sync_copy(sr0.at[pl.ds(0, 16)], cnts_h.at[pl.ds(w * 16, 16)])


def _k2(bo, src, dst, d, sw, species):
    nrec = NW * NB * CAP_RAD
    f = pl.kernel(
        _k2_body,
        out_type=(
            jax.ShapeDtypeStruct((nrec,), jnp.float32),
            jax.ShapeDtypeStruct((nrec,), jnp.float32),
            jax.ShapeDtypeStruct((nrec,), jnp.int32),
            jax.ShapeDtypeStruct((NW * 16,), jnp.int32),
        ),
        mesh=_MESH,
        compiler_params=pltpu.CompilerParams(needs_layout_passes=False, use_tc_tiling_on_sc=False),
        scratch_types=[
            pltpu.VMEM((C,), jnp.int32),
            pltpu.VMEM((C,), jnp.int32),
            pltpu.VMEM((C,), jnp.int32),
            pltpu.VMEM((C,), jnp.float32),
            pltpu.VMEM((C,), jnp.float32),
            pltpu.VMEM((C,), jnp.int32),
        ] + [pltpu.VMEM((F + 16,), jnp.float32) for _ in range(4)]
          + [pltpu.VMEM((F + 16,), jnp.float32) for _ in range(4)]
          + [pltpu.VMEM((F + 16,), jnp.int32) for _ in range(4)]
          + [pltpu.SemaphoreType.DMA],
    )
    return f(bo, src, dst, d, sw, species)


def _k3_body(ang_h, cen_h, asrc_h, adst_h, tab_h,
             ta_h, td_h, ts_h, trow_h, cnts_h,
             angv, cenv, asrcv, adstv, bufA, bufB,
             s0, s1, s2, s3, t0, t1, t2, t3, u0, u1, u2, u3, r0, r1, r2, r3,
             sem):
    w = _wid()
    nk = (CH_TRI - w + NW - 1) // NW
    sa = [s0, s1, s2, s3]
    sb = [t0, t1, t2, t3]
    sc_ = [u0, u1, u2, u3]
    sr = [r0, r1, r2, r3]

    def chunk(k, carry):
        c = w + k * NW
        off = jnp.minimum(c * C, N_ANG - C)
        lo = c * C
        pltpu.sync_copy(ang_h.at[pl.ds(off, C)], angv)
        pltpu.sync_copy(cen_h.at[pl.ds(off, C)], cenv)
        pltpu.sync_copy(asrc_h.at[pl.ds(off, C)], asrcv)
        pltpu.sync_copy(adst_h.at[pl.ds(off, C)], adstv)
        for j in range(C // 128):
            pltpu.async_copy(tab_h.at[asrcv.at[pl.ds(j * 128, 128)]],
                             bufA.at[pl.ds(j * 128, 128)], sem)
            pltpu.async_copy(tab_h.at[adstv.at[pl.ds(j * 128, 128)]],
                             bufB.at[pl.ds(j * 128, 128)], sem)
        for j in range(C // 128):
            pltpu.make_async_copy(tab_h.at[asrcv.at[pl.ds(j * 128, 128)]],
                                  bufA.at[pl.ds(j * 128, 128)], sem).wait()
            pltpu.make_async_copy(tab_h.at[adstv.at[pl.ds(j * 128, 128)]],
                                  bufB.at[pl.ds(j * 128, 128)], sem).wait()

        def grp(g, cr):
            row = g * 16 + _lane()
            gid = off + row
            valid = gid >= lo
            ang = plsc.load_gather(angv, [row])
            cen = plsc.load_gather(cenv, [row])
            dA = plsc.load_gather(bufA, [row, _splat(0)])
            swA = plsc.load_gather(bufA, [row, _splat(1)])
            spA = plsc.load_gather(bufA, [row, _splat(2)]).astype(jnp.int32)
            dB = plsc.load_gather(bufB, [row, _splat(0)])
            swB = plsc.load_gather(bufB, [row, _splat(1)])
            spB = plsc.load_gather(bufB, [row, _splat(2)]).astype(jnp.int32)
            d12 = 0.5 * (dA + dB)
            scale = 2.0 * swA * swB
            i = jnp.minimum(spA, spB)
            j_ = jnp.maximum(spA, spB)
            pair = i * 4 - (i * (i - 1)) // 2 + (j_ - i)
            bin_ = ((cen >= APB).astype(jnp.int32)
                    + (cen >= 2 * APB).astype(jnp.int32)
                    + (cen >= 3 * APB).astype(jnp.int32))
            trow = (cen - bin_ * APB) * 10 + pair
            cs = list(cr)
            for b in range(NB):
                m = jnp.logical_and(bin_ == b, valid)
                base = (w * NB + b) * CAP_ANG
                cs[b], cs[NB + b] = _append(
                    [sa[b], sb[b], sc_[b], sr[b]], [ang, d12, scale, trow], m,
                    cs[b], cs[NB + b], [ta_h, td_h, ts_h, trow_h], base)
            return tuple(cs)

        return lax.fori_loop(0, GROUPS, grp, carry)

    carry = lax.fori_loop(0, nk, chunk, (jnp.int32(0),) * (2 * NB))
    totals = [_drain([sa[b], sb[b], sc_[b], sr[b]], carry[b], carry[NB + b],
                     [ta_h, td_h, ts_h, trow_h], (w * NB + b) * CAP_ANG)
              for b in range(NB)]
    r0[pl.ds(0, 16)] = _counts_vec(totals)
    pltpu.sync_copy(r0.at[pl.ds(0, 16)], cnts_h.at[pl.ds(w * 16, 16)])


def _k3(ang, cen, asrc, adst, tab):
    nrec = NW * NB * CAP_ANG
    f = pl.kernel(
        _k3_body,
        out_type=(
            jax.ShapeDtypeStruct((nrec,), jnp.float32),
            jax.ShapeDtypeStruct((nrec,), jnp.float32),
            jax.ShapeDtypeStruct((nrec,), jnp.float32),
            jax.ShapeDtypeStruct((nrec,), jnp.int32),
            jax.ShapeDtypeStruct((NW * 16,), jnp.int32),
        ),
        mesh=_MESH,
        compiler_params=pltpu.CompilerParams(needs_layout_passes=False, use_tc_tiling_on_sc=False),
        scratch_types=[
            pltpu.VMEM((C,), jnp.float32),
            pltpu.VMEM((C,), jnp.int32),
            pltpu.VMEM((C,), jnp.int32),
            pltpu.VMEM((C,), jnp.int32),
            pltpu.VMEM((C, 16), jnp.float32),
            pltpu.VMEM((C, 16), jnp.float32),
        ] + [pltpu.VMEM((F + 16,), jnp.float32) for _ in range(12)]
          + [pltpu.VMEM((F + 16,), jnp.int32) for _ in range(4)]
          + [pltpu.SemaphoreType.DMA],
    )
    return f(ang, cen, asrc, adst, tab)


def _zero_fill(zbuf):
    zero16 = jnp.zeros((16,), jnp.float32)

    def zrow(i, _):
        plsc.store_scatter(zbuf, [_splat(i), _lane()], zero16)
        return 0
    lax.fori_loop(0, F, zrow, 0)


def _zero_or_flush(accum, hbm, real_rows, sub, zbuf, bucket_off, flush):
    nblk = real_rows // BROWS
    nbt = (nblk - sub + NS - 1) // NS

    def blk(i, _):
        o = (sub + i * NS) * BROWS
        if flush:
            pltpu.sync_copy(accum.at[pl.ds(o, BROWS)],
                            hbm.at[pl.ds(bucket_off + o, BROWS)])
        else:
            pltpu.sync_copy(zbuf.at[pl.ds(0, BROWS)], accum.at[pl.ds(o, BROWS)])
        return 0

    lax.fori_loop(0, nbt, blk, 0)


def _k4_body(rd_h, rsw_h, rrow_h, cnts_h, acc_out,
             accum, dbuf, swbuf, rowraw, idxbuf, valbuf, zbuf, cntv):
    core = lax.axis_index("c")
    sub = lax.axis_index("s")
    _zero_fill(zbuf)

    for p in range(2):
        bucket = core * 2 + p
        _zero_or_flush(accum, acc_out, R_RAD, sub, zbuf, 0, False)
        plsc.subcore_barrier()

        for s in range(2):
            wsrc = sub * 2 + s
            pltpu.sync_copy(cnts_h.at[pl.ds(wsrc * 16, 16)], cntv)
            cnt = _extract(cntv[...], bucket)
            base = (wsrc * NB + bucket) * CAP_RAD
            nch = (cnt + F - 1) // F

            def chunk(k, _):
                offr = base + k * F
                pltpu.sync_copy(rd_h.at[pl.ds(offr, F)], dbuf)
                pltpu.sync_copy(rsw_h.at[pl.ds(offr, F)], swbuf)
                pltpu.sync_copy(rrow_h.at[pl.ds(offr, F)], rowraw)
                for gg in range(8):
                    sl = gg * 16
                    d = dbuf[pl.ds(sl, 16)]
                    sw = swbuf[pl.ds(sl, 16)]
                    rrow = rowraw[pl.ds(sl, 16)]
                    slot = k * F + sl + _lane()
                    rrow = jnp.where(slot < cnt, rrow, _splat(GARB_RAD))
                    idxbuf[pl.ds(sl, 16)] = rrow
                    pre = 0.25 * sw
                    rvec = sl + _lane()
                    for j in range(16):
                        x = d - _SHIFT_R[j]
                        t = jnp.exp(x * x * (-16.0)) * pre
                        plsc.store_scatter(valbuf, [rvec, _splat(j)], t)
                pltpu.sync_copy(valbuf, accum.at[idxbuf], add=True)
                return 0

            lax.fori_loop(0, nch, chunk, 0)
        plsc.subcore_barrier()
        _zero_or_flush(accum, acc_out, R_RAD, sub, zbuf, bucket * R_RAD, True)
        plsc.subcore_barrier()


def _k4(rd, rsw, rrow, cnts):
    f = pl.kernel(
        _k4_body,
        out_type=jax.ShapeDtypeStruct((NB * R_RAD, 16), jnp.float32),
        mesh=_MESH,
        compiler_params=pltpu.CompilerParams(needs_layout_passes=False, use_tc_tiling_on_sc=False),
        scratch_types=[
            pltpu.VMEM_SHARED((RA_RAD, 16), jnp.float32),
            pltpu.VMEM((F,), jnp.float32),
            pltpu.VMEM((F,), jnp.float32),
            pltpu.VMEM((F,), jnp.int32),
            pltpu.VMEM((F,), jnp.int32),
            pltpu.VMEM((F, 16), jnp.float32),
            pltpu.VMEM((F, 16), jnp.float32),
            pltpu.VMEM((16,), jnp.int32),
        ],
    )
    return f(rd, rsw, rrow, cnts)


def _k5_body(ta_h, td_h, ts_h, trow_h, cnts_h, acc_out,
             accum, abuf, dbuf, sbuf, rowraw, idxbuf, valbuf, zbuf, cntv):
    core = lax.axis_index("c")
    sub = lax.axis_index("s")
    _zero_fill(zbuf)

    for p in range(2):
        bucket = core * 2 + p
        _zero_or_flush(accum, acc_out, R_ANG, sub, zbuf, 0, False)
        plsc.subcore_barrier()

        for s in range(2):
            wsrc = sub * 2 + s
            pltpu.sync_copy(cnts_h.at[pl.ds(wsrc * 16, 16)], cntv)
            cnt = _extract(cntv[...], bucket)
            base = (wsrc * NB + bucket) * CAP_ANG
            nch = (cnt + F - 1) // F

            def chunk(k, _):
                offr = base + k * F
                pltpu.sync_copy(ta_h.at[pl.ds(offr, F)], abuf)
                pltpu.sync_copy(td_h.at[pl.ds(offr, F)], dbuf)
                pltpu.sync_copy(ts_h.at[pl.ds(offr, F)], sbuf)
                pltpu.sync_copy(trow_h.at[pl.ds(offr, F)], rowraw)
                for gg in range(8):
                    sl = gg * 16
                    ang = abuf[pl.ds(sl, 16)]
                    d12 = dbuf[pl.ds(sl, 16)]
                    scale = sbuf[pl.ds(sl, 16)]
                    trow = rowraw[pl.ds(sl, 16)]
                    slot = k * F + sl + _lane()
                    trow = jnp.where(slot < cnt, trow, _splat(GARB_ANG))
                    idxbuf[pl.ds(sl, 16)] = trow
                    u = ang - _HALF_PI
                    t2 = u * u
                    sinu = u * (_SINP[0] + t2 * (_SINP[1] + t2 * (_SINP[2] + t2 * _SINP[3])))
                    cosu = (_COSP[0] + t2 * (_COSP[1] + t2 * (_COSP[2]
                            + t2 * (_COSP[3] + t2 * _COSP[4]))))
                    cth = -sinu
                    sth = cosu
                    y32 = []
                    for z in range(4):
                        y = 0.5 + 0.5 * (cth * _CZ[z] + sth * _SZ[z])
                        y = y * y
                        y = y * y
                        y = y * y
                        y = y * y
                        y = y * y
                        y32.append(y)
                    ea = []
                    for a in range(4):
                        x = d12 - _SHIFT_A[a]
                        ea.append(jnp.exp(x * x * (-8.0)) * scale)
                    rvec = sl + _lane()
                    for fno in range(16):
                        v = y32[fno % 4] * ea[fno // 4]
                        plsc.store_scatter(valbuf, [rvec, _splat(fno)], v)
                pltpu.sync_copy(valbuf, accum.at[idxbuf], add=True)
                return 0

            lax.fori_loop(0, nch, chunk, 0)
        plsc.subcore_barrier()
        _zero_or_flush(accum, acc_out, R_ANG, sub, zbuf, bucket * R_ANG, True)
        plsc.subcore_barrier()


def _k5(ta, td, ts, trow, cnts):
    f = pl.kernel(
        _k5_body,
        out_type=jax.ShapeDtypeStruct((NB * R_ANG, 16), jnp.float32),
        mesh=_MESH,
        compiler_params=pltpu.CompilerParams(needs_layout_passes=False, use_tc_tiling_on_sc=False),
        scratch_types=[
            pltpu.VMEM_SHARED((RA_ANG, 16), jnp.float32),
            pltpu.VMEM((F,), jnp.float32),
            pltpu.VMEM((F,), jnp.float32),
            pltpu.VMEM((F,), jnp.float32),
            pltpu.VMEM((F,), jnp.int32),
            pltpu.VMEM((F,), jnp.int32),
            pltpu.VMEM((F, 16), jnp.float32),
            pltpu.VMEM((F, 16), jnp.float32),
            pltpu.VMEM((16,), jnp.int32),
        ],
    )
    return f(ta, td, ts, trow, cnts)


def _k6_body(ecfp_ref, rad_ref, ang_ref, out_ref):
    e = ecfp_ref[...]
    r = rad_ref[0]
    a = ang_ref[0]
    i_ = jax.lax.broadcasted_iota(jnp.int32, (128, 128), 0)
    o_ = jax.lax.broadcasted_iota(jnp.int32, (128, 128), 1)
    perm = (i_ // 32) * 32 + (i_ % 16) * 2 + (i_ // 16) % 2
    P = jnp.where(o_ == perm, 1.0, 0.0).astype(jnp.float32)
    rp = jax.lax.dot(r, P, precision=jax.lax.Precision.HIGHEST)
    out_ref[...] = jnp.concatenate((e, rp, a), axis=1)


def _k6(ecfp, rad_acc, ang_acc):
    nblk = N_ATOMS // 80
    rad3 = rad_acc.reshape(nblk, 80, 128)
    ang3 = ang_acc.reshape(nblk, 80, 160)
    f = pl.pallas_call(
        _k6_body,
        grid=(nblk,),
        in_specs=[
            pl.BlockSpec((80, 16), lambda i: (i, 0)),
            pl.BlockSpec((1, 80, 128), lambda i: (i, 0, 0)),
            pl.BlockSpec((1, 80, 160), lambda i: (i, 0, 0)),
        ],
        out_specs=pl.BlockSpec((80, 304), lambda i: (i, 0)),
        out_shape=jax.ShapeDtypeStruct((N_ATOMS, 304), jnp.float32),
    )
    return f(ecfp, rad3, ang3)


def kernel(species, ecfp, bond_order, edge_src, edge_dst, distances, switch,
           angles, ang_distances, central_atom, angle_src, angle_dst,
           ang_switch, ang_edge_dst):
    tab = _k1(ang_distances, ang_switch, ang_edge_dst, species)
    rd, rsw, rrow, rcnts = _k2(bond_order, edge_src, edge_dst, distances,
                               switch, species)
    ta, td, ts, trow, tcnts = _k3(angles, central_atom, angle_src, angle_dst, tab)
    rad_acc = _k4(rd, rsw, rrow, rcnts)
    ang_acc = _k5(ta, td, ts, trow, tcnts)
    return _k6(ecfp, rad_acc, ang_acc)


# speedup vs baseline: 79.2731x; 1.1988x over previous
import functools
import numpy as np
import jax
import jax.numpy as jnp
from jax import lax
from jax.experimental import pallas as pl
from jax.experimental.pallas import tpu as pltpu
from jax.experimental.pallas import tpu_sc as plsc

NC, NS, L = 2, 16, 16
NW = NC * NS

N_ATOMS = 50000
E_RAD = 800000
E_ANG = 400000
N_ANG = 1400000
NB = 4
APB = N_ATOMS // NB

C = 1536
GROUPS = C // 16
CH_RAD = -(-E_RAD // C)
CH_TRI = -(-N_ANG // C)
C3 = 1024
CH_TRI3 = -(-N_ANG // C3)
CH_TAB = -(-E_ANG // C)
CAP_RAD = 26624
CAP_ANG = 44544
F = 128
CHREC = 512
RECPAD = 512

R_RAD = 100000
RA_RAD = 100016
GARB_RAD = 100000
NB_ANG = 6
APB_ANG = 8400
R_ANG = 84000
R_ANG_LAST = 80000
RA_ANG = 84008
GARB_ANG = 84000
BROWS = 125

_SINP = (9.9999924135e-01, -1.6665679619e-01, 8.3132250799e-03, -1.8523448330e-04)
_COSP = (9.9999995325e-01, -4.9999905063e-01, 4.1663578931e-02, -1.3853666933e-03,
         2.3153174156e-05)
_HALF_PI = float(np.pi / 2)
_SHIFT_Z = [float(np.pi / 8 + k * np.pi / 4) for k in range(4)]
_CZ = [float(np.cos(z)) for z in _SHIFT_Z]
_SZ = [float(np.sin(z)) for z in _SHIFT_Z]
_SHIFT_A = [0.8 + 0.675 * a for a in range(4)]
_SHIFT_R = [0.8 + 0.275 * j for j in range(16)]

_MESH = plsc.VectorSubcoreMesh(core_axis_name="c", subcore_axis_name="s",
                               num_cores=NC, num_subcores=NS)


def _lane():
    return lax.iota(jnp.int32, 16)


def _splat(x, dtype=jnp.int32):
    return jnp.full((16,), x, dtype=dtype)


def _wid():
    return lax.axis_index("s") * NC + lax.axis_index("c")


def _extract(vec16, pos):
    return jnp.sum(jnp.where(_lane() == pos, vec16, 0))


def _spec_idx(s):
    return ((s == 6).astype(jnp.int32) + 2 * (s == 7).astype(jnp.int32)
            + 3 * (s == 8).astype(jnp.int32))


def _append(stags, fields, mask, cnt, kb, outs, base):
    n = jnp.sum(mask.astype(jnp.int32))
    for sref, f in zip(stags, fields):
        plsc.store_compressed(sref.at[pl.ds(cnt, 16)], f, mask=mask)
    new = cnt + n
    do = new >= F

    @pl.when(do)
    def _():
        for sref, oref in zip(stags, outs):
            pltpu.sync_copy(sref.at[pl.ds(0, F)], oref.at[pl.ds(base + kb * F, F)])
            sref[pl.ds(0, 16)] = sref[pl.ds(F, 16)]

    doi = do.astype(jnp.int32)
    return new - F * doi, kb + doi


def _drain(stags, cnt, kb, outs, base):
    @pl.when(cnt > 0)
    def _():
        for sref, oref in zip(stags, outs):
            pltpu.sync_copy(sref.at[pl.ds(0, F)], oref.at[pl.ds(base + kb * F, F)])
    return kb * F + cnt


def _counts_vec(totals):
    v = _splat(0)
    for b, t in enumerate(totals):
        v = jnp.where(_lane() == b, _splat(t), v)
    return v


def _k1_body(angd, angsw, angdst, species, tab, dv, swv, dstv, spv, stag, sem):
    w = _wid()
    nk = (CH_TAB - w + NW - 1) // NW

    def chunk(k, carry):
        c = w + k * NW
        off = jnp.minimum(c * C, E_ANG - C)
        pltpu.sync_copy(angd.at[pl.ds(off, C)], dv)
        pltpu.sync_copy(angsw.at[pl.ds(off, C)], swv)
        pltpu.sync_copy(angdst.at[pl.ds(off, C)], dstv)
        for j in range(C // 128):
            pltpu.async_copy(species.at[dstv.at[pl.ds(j * 128, 128)]],
                             spv.at[pl.ds(j * 128, 128)], sem)
        for j in range(C // 128):
            pltpu.make_async_copy(species.at[dstv.at[pl.ds(j * 128, 128)]],
                                  spv.at[pl.ds(j * 128, 128)], sem).wait()

        def grp(g, _):
            row = g * 16 + _lane()
            d = plsc.load_gather(dv, [row])
            sw = plsc.load_gather(swv, [row])
            sp = plsc.load_gather(spv, [row])
            spf = _spec_idx(sp).astype(jnp.float32)
            plsc.store_scatter(stag, [row, _splat(0)], d)
            plsc.store_scatter(stag, [row, _splat(1)], sw)
            plsc.store_scatter(stag, [row, _splat(2)], spf)
            return 0

        lax.fori_loop(0, GROUPS, grp, 0)
        pltpu.sync_copy(stag, tab.at[pl.ds(off, C)])
        return 0

    lax.fori_loop(0, nk, chunk, 0)


def _k1(angd, angsw, angdst, species):
    f = pl.kernel(
        _k1_body,
        out_type=jax.ShapeDtypeStruct((E_ANG, 16), jnp.float32),
        mesh=_MESH,
        compiler_params=pltpu.CompilerParams(needs_layout_passes=False, use_tc_tiling_on_sc=False),
        scratch_types=[
            pltpu.VMEM((C,), jnp.float32),
            pltpu.VMEM((C,), jnp.float32),
            pltpu.VMEM((C,), jnp.int32),
            pltpu.VMEM((C,), jnp.int32),
            pltpu.VMEM((C, 16), jnp.float32),
            pltpu.SemaphoreType.DMA,
        ],
    )
    return f(angd, angsw, angdst, species)


def _k2_body(bo_h, src_h, dst_h, d_h, sw_h, species,
             rd_h, rsw_h, rrow_h, cnts_h,
             bov, srcv, dstv, dv, swv, spv, sd0, sd1, sd2, sd3,
             ss0, ss1, ss2, ss3, sr0, sr1, sr2, sr3, sem):
    w = _wid()
    nk = (CH_RAD - w + NW - 1) // NW
    sd = [sd0, sd1, sd2, sd3]
    ss = [ss0, ss1, ss2, ss3]
    sr = [sr0, sr1, sr2, sr3]

    def chunk(k, carry):
        c = w + k * NW
        off = jnp.minimum(c * C, E_RAD - C)
        lo = c * C
        pltpu.sync_copy(bo_h.at[pl.ds(off, C)], bov)
        pltpu.sync_copy(src_h.at[pl.ds(off, C)], srcv)
        pltpu.sync_copy(dst_h.at[pl.ds(off, C)], dstv)
        pltpu.sync_copy(d_h.at[pl.ds(off, C)], dv)
        pltpu.sync_copy(sw_h.at[pl.ds(off, C)], swv)
        for j in range(C // 128):
            pltpu.async_copy(species.at[dstv.at[pl.ds(j * 128, 128)]],
                             spv.at[pl.ds(j * 128, 128)], sem)
        for j in range(C // 128):
            pltpu.make_async_copy(species.at[dstv.at[pl.ds(j * 128, 128)]],
                                  spv.at[pl.ds(j * 128, 128)], sem).wait()

        def grp(g, cr):
            row = g * 16 + _lane()
            gid = off + row
            valid = gid >= lo
            bo = plsc.load_gather(bov, [row])
            src = plsc.load_gather(srcv, [row])
            d = plsc.load_gather(dv, [row])
            sw = plsc.load_gather(swv, [row])
            sp = _spec_idx(plsc.load_gather(spv, [row]))
            ch = (jnp.logical_or(bo == 3, bo == 5)).astype(jnp.int32)
            bin_ = ((src >= APB).astype(jnp.int32)
                    + (src >= 2 * APB).astype(jnp.int32)
                    + (src >= 3 * APB).astype(jnp.int32))
            rrow = (src - bin_ * APB) * 8 + sp * 2 + ch
            cs = list(cr)
            for b in range(NB):
                m = jnp.logical_and(bin_ == b, valid)
                base = (w * NB + b) * CAP_RAD
                cs[b], cs[NB + b] = _append(
                    [sd[b], ss[b], sr[b]], [d, sw, rrow], m,
                    cs[b], cs[NB + b], [rd_h, rsw_h, rrow_h], base)
            return tuple(cs)

        return lax.fori_loop(0, GROUPS, grp, carry)

    carry = lax.fori_loop(0, nk, chunk, (jnp.int32(0),) * (2 * NB))
    totals = [_drain([sd[b], ss[b], sr[b]], carry[b], carry[NB + b],
                     [rd_h, rsw_h, rrow_h], (w * NB + b) * CAP_RAD)
              for b in range(NB)]
    sr0[pl.ds(0, 16)] = _counts_vec(totals)
    pltpu.sync_copy(sr0.at[pl.ds(0, 16)], cnts_h.at[pl.ds(w * 16, 16)])


def _k2(bo, src, dst, d, sw, species):
    nrec = NW * NB * CAP_RAD + RECPAD
    f = pl.kernel(
        _k2_body,
        out_type=(
            jax.ShapeDtypeStruct((nrec,), jnp.float32),
            jax.ShapeDtypeStruct((nrec,), jnp.float32),
            jax.ShapeDtypeStruct((nrec,), jnp.int32),
            jax.ShapeDtypeStruct((NW * 16,), jnp.int32),
        ),
        mesh=_MESH,
        compiler_params=pltpu.CompilerParams(needs_layout_passes=False, use_tc_tiling_on_sc=False),
        scratch_types=[
            pltpu.VMEM((C,), jnp.int32),
            pltpu.VMEM((C,), jnp.int32),
            pltpu.VMEM((C,), jnp.int32),
            pltpu.VMEM((C,), jnp.float32),
            pltpu.VMEM((C,), jnp.float32),
            pltpu.VMEM((C,), jnp.int32),
        ] + [pltpu.VMEM((F + 16,), jnp.float32) for _ in range(4)]
          + [pltpu.VMEM((F + 16,), jnp.float32) for _ in range(4)]
          + [pltpu.VMEM((F + 16,), jnp.int32) for _ in range(4)]
          + [pltpu.SemaphoreType.DMA],
    )
    return f(bo, src, dst, d, sw, species)


def _k3_body(ang_h, cen_h, asrc_h, adst_h, tab_h,
             ta_h, td_h, ts_h, trow_h, cnts_h, *scr):
    angv = scr[0:2]
    cenv = scr[2:4]
    asrcv = scr[4:6]
    adstv = scr[6:8]
    bufA = scr[8:10]
    bufB = scr[10:12]
    rest = scr[12:-2]
    sems = scr[-2:]
    sa = list(rest[0:NB_ANG])
    sb = list(rest[NB_ANG:2 * NB_ANG])
    sc_ = list(rest[2 * NB_ANG:3 * NB_ANG])
    sr = list(rest[3 * NB_ANG:4 * NB_ANG])
    w = _wid()
    nk = (CH_TRI3 - w + NW - 1) // NW

    def fire(k, par):
        c = w + k * NW
        off = jnp.minimum(c * C3, N_ANG - C3)
        pltpu.sync_copy(asrc_h.at[pl.ds(off, C3)], asrcv[par])
        pltpu.sync_copy(adst_h.at[pl.ds(off, C3)], adstv[par])
        pltpu.sync_copy(ang_h.at[pl.ds(off, C3)], angv[par])
        pltpu.sync_copy(cen_h.at[pl.ds(off, C3)], cenv[par])
        for j in range(C3 // 128):
            pltpu.async_copy(tab_h.at[asrcv[par].at[pl.ds(j * 128, 128)]],
                             bufA[par].at[pl.ds(j * 128, 128)], sems[par])
            pltpu.async_copy(tab_h.at[adstv[par].at[pl.ds(j * 128, 128)]],
                             bufB[par].at[pl.ds(j * 128, 128)], sems[par])

    def wait(par):
        for j in range(C3 // 128):
            pltpu.make_async_copy(tab_h.at[asrcv[par].at[pl.ds(j * 128, 128)]],
                                  bufA[par].at[pl.ds(j * 128, 128)],
                                  sems[par]).wait()
            pltpu.make_async_copy(tab_h.at[adstv[par].at[pl.ds(j * 128, 128)]],
                                  bufB[par].at[pl.ds(j * 128, 128)],
                                  sems[par]).wait()

    def process(k, par, carry):
        c = w + k * NW
        off = jnp.minimum(c * C3, N_ANG - C3)
        lo = c * C3

        def grp(g, cr):
            row = g * 16 + _lane()
            gid = off + row
            valid = gid >= lo
            ang = plsc.load_gather(angv[par], [row])
            cen = plsc.load_gather(cenv[par], [row])
            dA = plsc.load_gather(bufA[par], [row, _splat(0)])
            swA = plsc.load_gather(bufA[par], [row, _splat(1)])
            spA = plsc.load_gather(bufA[par], [row, _splat(2)]).astype(jnp.int32)
            dB = plsc.load_gather(bufB[par], [row, _splat(0)])
            swB = plsc.load_gather(bufB[par], [row, _splat(1)])
            spB = plsc.load_gather(bufB[par], [row, _splat(2)]).astype(jnp.int32)
            d12 = 0.5 * (dA + dB)
            scale = 2.0 * swA * swB
            i = jnp.minimum(spA, spB)
            j_ = jnp.maximum(spA, spB)
            pair = i * 4 - (i * (i - 1)) // 2 + (j_ - i)
            bin_ = (cen >= APB_ANG).astype(jnp.int32)
            for bb in range(2, NB_ANG):
                bin_ = bin_ + (cen >= bb * APB_ANG).astype(jnp.int32)
            trow = (cen - bin_ * APB_ANG) * 10 + pair
            cs = list(cr)
            for b in range(NB_ANG):
                m = jnp.logical_and(bin_ == b, valid)
                base = (w * NB_ANG + b) * CAP_ANG
                cs[b], cs[NB_ANG + b] = _append(
                    [sa[b], sb[b], sc_[b], sr[b]], [ang, d12, scale, trow], m,
                    cs[b], cs[NB_ANG + b], [ta_h, td_h, ts_h, trow_h], base)
            return tuple(cs)

        return lax.fori_loop(0, C3 // 16, grp, carry)

    fire(jnp.int32(0), 0)

    def pair_body(kp, carry):
        out = carry
        for par in (0, 1):
            k = kp * 2 + par

            def do(cr):
                @pl.when(k + 1 < nk)
                def _():
                    fire(k + 1, 1 - par)
                wait(par)
                return process(k, par, cr)

            out = lax.cond(k < nk, do, lambda cr: cr, out)
        return out

    carry = lax.fori_loop(0, (nk + 1) // 2, pair_body,
                          (jnp.int32(0),) * (2 * NB_ANG))
    totals = [_drain([sa[b], sb[b], sc_[b], sr[b]], carry[b], carry[NB_ANG + b],
                     [ta_h, td_h, ts_h, trow_h], (w * NB_ANG + b) * CAP_ANG)
              for b in range(NB_ANG)]
    sr[0][pl.ds(0, 16)] = _counts_vec(totals)
    pltpu.sync_copy(sr[0].at[pl.ds(0, 16)], cnts_h.at[pl.ds(w * 16, 16)])


def _k3(ang, cen, asrc, adst, tab):
    nrec = NW * NB_ANG * CAP_ANG + RECPAD
    f = pl.kernel(
        _k3_body,
        out_type=(
            jax.ShapeDtypeStruct((nrec,), jnp.float32),
            jax.ShapeDtypeStruct((nrec,), jnp.float32),
            jax.ShapeDtypeStruct((nrec,), jnp.float32),
            jax.ShapeDtypeStruct((nrec,), jnp.int32),
            jax.ShapeDtypeStruct((NW * 16,), jnp.int32),
        ),
        mesh=_MESH,
        compiler_params=pltpu.CompilerParams(needs_layout_passes=False, use_tc_tiling_on_sc=False),
        scratch_types=[
            pltpu.VMEM((C3,), jnp.float32),
            pltpu.VMEM((C3,), jnp.float32),
            pltpu.VMEM((C3,), jnp.int32),
            pltpu.VMEM((C3,), jnp.int32),
            pltpu.VMEM((C3,), jnp.int32),
            pltpu.VMEM((C3,), jnp.int32),
            pltpu.VMEM((C3,), jnp.int32),
            pltpu.VMEM((C3,), jnp.int32),
            pltpu.VMEM((C3, 16), jnp.float32),
            pltpu.VMEM((C3, 16), jnp.float32),
            pltpu.VMEM((C3, 16), jnp.float32),
            pltpu.VMEM((C3, 16), jnp.float32),
        ] + [pltpu.VMEM((F + 16,), jnp.float32) for _ in range(3 * NB_ANG)]
          + [pltpu.VMEM((F + 16,), jnp.int32) for _ in range(NB_ANG)]
          + [pltpu.SemaphoreType.DMA, pltpu.SemaphoreType.DMA],
    )
    return f(ang, cen, asrc, adst, tab)


def _zero_fill(zbuf):
    zero16 = jnp.zeros((16,), jnp.float32)

    def zrow(i, _):
        plsc.store_scatter(zbuf, [_splat(i), _lane()], zero16)
        return 0
    lax.fori_loop(0, F, zrow, 0)


def _zero_or_flush(accum, hbm, real_rows, sub, zbuf, bucket_off, flush):
    nblk = real_rows // BROWS
    nbt = (nblk - sub + NS - 1) // NS

    def blk(i, _):
        o = (sub + i * NS) * BROWS
        if flush:
            pltpu.sync_copy(accum.at[pl.ds(o, BROWS)],
                            hbm.at[pl.ds(bucket_off + o, BROWS)])
        else:
            pltpu.sync_copy(zbuf.at[pl.ds(0, BROWS)], accum.at[pl.ds(o, BROWS)])
        return 0

    lax.fori_loop(0, nbt, blk, 0)


def _k4_body(rd_h, rsw_h, rrow_h, cnts_h, acc_out,
             accum, dbuf, swbuf, rowraw, idxbuf, valbuf, zbuf, cntv, sem):
    core = lax.axis_index("c")
    sub = lax.axis_index("s")
    _zero_fill(zbuf)

    for p in range(2):
        bucket = core * 2 + p
        _zero_or_flush(accum, acc_out, R_RAD, sub, zbuf, 0, False)
        plsc.subcore_barrier()

        for s in range(2):
            wsrc = sub * 2 + s
            pltpu.sync_copy(cnts_h.at[pl.ds(wsrc * 16, 16)], cntv)
            cnt = _extract(cntv[...], bucket)
            base = (wsrc * NB + bucket) * CAP_RAD
            nch = (cnt + CHREC - 1) // CHREC

            def chunk(k, _):
                offr = base + k * CHREC
                pltpu.sync_copy(rd_h.at[pl.ds(offr, CHREC)], dbuf)
                pltpu.sync_copy(rsw_h.at[pl.ds(offr, CHREC)], swbuf)
                pltpu.sync_copy(rrow_h.at[pl.ds(offr, CHREC)], rowraw)
                for sc in range(CHREC // F):
                    for gg in range(8):
                        sl = sc * F + gg * 16
                        d = dbuf[pl.ds(sl, 16)]
                        sw = swbuf[pl.ds(sl, 16)]
                        rrow = rowraw[pl.ds(sl, 16)]
                        slot = k * CHREC + sl + _lane()
                        rrow = jnp.where(slot < cnt, rrow, _splat(GARB_RAD))
                        plsc.store_scatter(idxbuf, [_splat(sc), gg * 16 + _lane()],
                                           rrow)
                        pre = 0.25 * sw
                        rvec = sl + _lane()
                        for j in range(16):
                            x = d - _SHIFT_R[j]
                            t = jnp.exp(x * x * (-16.0)) * pre
                            plsc.store_scatter(valbuf, [rvec, _splat(j)], t)
                    pltpu.async_copy(valbuf.at[pl.ds(sc * F, F)],
                                     accum.at[idxbuf.at[sc]], sem, add=True)
                for sc in range(CHREC // F):
                    pltpu.make_async_copy(valbuf.at[pl.ds(sc * F, F)],
                                          accum.at[idxbuf.at[sc]], sem).wait()
                return 0

            lax.fori_loop(0, nch, chunk, 0)
        plsc.subcore_barrier()
        _zero_or_flush(accum, acc_out, R_RAD, sub, zbuf, bucket * R_RAD, True)
        plsc.subcore_barrier()


def _k4(rd, rsw, rrow, cnts):
    f = pl.kernel(
        _k4_body,
        out_type=jax.ShapeDtypeStruct((NB * R_RAD, 16), jnp.float32),
        mesh=_MESH,
        compiler_params=pltpu.CompilerParams(needs_layout_passes=False, use_tc_tiling_on_sc=False),
        scratch_types=[
            pltpu.VMEM_SHARED((RA_RAD, 16), jnp.float32),
            pltpu.VMEM((CHREC,), jnp.float32),
            pltpu.VMEM((CHREC,), jnp.float32),
            pltpu.VMEM((CHREC,), jnp.int32),
            pltpu.VMEM((CHREC // F, F), jnp.int32),
            pltpu.VMEM((CHREC, 16), jnp.float32),
            pltpu.VMEM((F, 16), jnp.float32),
            pltpu.VMEM((16,), jnp.int32),
            pltpu.SemaphoreType.DMA,
        ],
    )
    return f(rd, rsw, rrow, cnts)


def _k5_body(ta_h, td_h, ts_h, trow_h, cnts_h, acc_out,
             accum, abuf, dbuf, sbuf, rowraw, idxbuf, valbuf, zbuf, cntv, sem):
    core = lax.axis_index("c")
    sub = lax.axis_index("s")
    _zero_fill(zbuf)

    def pass_body(p, _):
        bucket = core * 3 + p
        real_rows = jnp.where(bucket == NB_ANG - 1, R_ANG_LAST, R_ANG)
        _zero_or_flush(accum, acc_out, R_ANG, sub, zbuf, 0, False)
        plsc.subcore_barrier()

        for s in range(2):
            wsrc = sub * 2 + s
            pltpu.sync_copy(cnts_h.at[pl.ds(wsrc * 16, 16)], cntv)
            cnt = _extract(cntv[...], bucket)
            base = (wsrc * NB_ANG + bucket) * CAP_ANG
            nch = (cnt + CHREC - 1) // CHREC

            def chunk(k, _):
                offr = base + k * CHREC
                pltpu.sync_copy(ta_h.at[pl.ds(offr, CHREC)], abuf)
                pltpu.sync_copy(td_h.at[pl.ds(offr, CHREC)], dbuf)
                pltpu.sync_copy(ts_h.at[pl.ds(offr, CHREC)], sbuf)
                pltpu.sync_copy(trow_h.at[pl.ds(offr, CHREC)], rowraw)
                for sc in range(CHREC // F):
                    for gg in range(8):
                        sl = sc * F + gg * 16
                        ang = abuf[pl.ds(sl, 16)]
                        d12 = dbuf[pl.ds(sl, 16)]
                        scale = sbuf[pl.ds(sl, 16)]
                        trow = rowraw[pl.ds(sl, 16)]
                        slot = k * CHREC + sl + _lane()
                        trow = jnp.where(slot < cnt, trow, _splat(GARB_ANG))
                        plsc.store_scatter(idxbuf, [_splat(sc), gg * 16 + _lane()],
                                           trow)
                        u = ang - _HALF_PI
                        t2 = u * u
                        sinu = u * (_SINP[0] + t2 * (_SINP[1] + t2 * (_SINP[2] + t2 * _SINP[3])))
                        cosu = (_COSP[0] + t2 * (_COSP[1] + t2 * (_COSP[2]
                                + t2 * (_COSP[3] + t2 * _COSP[4]))))
                        cth = -sinu
                        sth = cosu
                        y32 = []
                        for z in range(4):
                            y = 0.5 + 0.5 * (cth * _CZ[z] + sth * _SZ[z])
                            y = y * y
                            y = y * y
                            y = y * y
                            y = y * y
                            y = y * y
                            y32.append(y)
                        ea = []
                        for a in range(4):
                            x = d12 - _SHIFT_A[a]
                            ea.append(jnp.exp(x * x * (-8.0)) * scale)
                        rvec = sl + _lane()
                        for fno in range(16):
                            v = y32[fno % 4] * ea[fno // 4]
                            plsc.store_scatter(valbuf, [rvec, _splat(fno)], v)
                    pltpu.async_copy(valbuf.at[pl.ds(sc * F, F)],
                                     accum.at[idxbuf.at[sc]], sem, add=True)
                for sc in range(CHREC // F):
                    pltpu.make_async_copy(valbuf.at[pl.ds(sc * F, F)],
                                          accum.at[idxbuf.at[sc]], sem).wait()
                return 0

            lax.fori_loop(0, nch, chunk, 0)
        plsc.subcore_barrier()
        _zero_or_flush(accum, acc_out, real_rows, sub, zbuf, bucket * R_ANG, True)
        plsc.subcore_barrier()
        return 0

    lax.fori_loop(0, 3, pass_body, 0)


def _k5(ta, td, ts, trow, cnts):
    f = pl.kernel(
        _k5_body,
        out_type=jax.ShapeDtypeStruct((N_ATOMS * 10, 16), jnp.float32),
        mesh=_MESH,
        compiler_params=pltpu.CompilerParams(needs_layout_passes=False, use_tc_tiling_on_sc=False),
        scratch_types=[
            pltpu.VMEM_SHARED((RA_ANG, 16), jnp.float32),
            pltpu.VMEM((CHREC,), jnp.float32),
            pltpu.VMEM((CHREC,), jnp.float32),
            pltpu.VMEM((CHREC,), jnp.float32),
            pltpu.VMEM((CHREC,), jnp.int32),
            pltpu.VMEM((CHREC // F, F), jnp.int32),
            pltpu.VMEM((CHREC, 16), jnp.float32),
            pltpu.VMEM((F, 16), jnp.float32),
            pltpu.VMEM((16,), jnp.int32),
            pltpu.SemaphoreType.DMA,
        ],
    )
    return f(ta, td, ts, trow, cnts)


def _k6_body(ecfp_ref, rad_ref, ang_ref, out_ref):
    e = ecfp_ref[...]
    r = rad_ref[0]
    a = ang_ref[0]
    i_ = jax.lax.broadcasted_iota(jnp.int32, (128, 128), 0)
    o_ = jax.lax.broadcasted_iota(jnp.int32, (128, 128), 1)
    perm = (i_ // 32) * 32 + (i_ % 16) * 2 + (i_ // 16) % 2
    P = jnp.where(o_ == perm, 1.0, 0.0).astype(jnp.float32)
    rp = jax.lax.dot(r, P, precision=jax.lax.Precision.HIGHEST)
    out_ref[...] = jnp.concatenate((e, rp, a), axis=1)


def _k6(ecfp, rad_acc, ang_acc):
    nblk = N_ATOMS // 80
    rad3 = rad_acc.reshape(nblk, 80, 128)
    ang3 = ang_acc.reshape(nblk, 80, 160)
    f = pl.pallas_call(
        _k6_body,
        grid=(nblk,),
        in_specs=[
            pl.BlockSpec((80, 16), lambda i: (i, 0)),
            pl.BlockSpec((1, 80, 128), lambda i: (i, 0, 0)),
            pl.BlockSpec((1, 80, 160), lambda i: (i, 0, 0)),
        ],
        out_specs=pl.BlockSpec((80, 304), lambda i: (i, 0)),
        out_shape=jax.ShapeDtypeStruct((N_ATOMS, 304), jnp.float32),
    )
    return f(ecfp, rad3, ang3)


def kernel(species, ecfp, bond_order, edge_src, edge_dst, distances, switch,
           angles, ang_distances, central_atom, angle_src, angle_dst,
           ang_switch, ang_edge_dst):
    tab = _k1(ang_distances, ang_switch, ang_edge_dst, species)
    rd, rsw, rrow, rcnts = _k2(bond_order, edge_src, edge_dst, distances,
                               switch, species)
    ta, td, ts, trow, tcnts = _k3(angles, central_atom, angle_src, angle_dst, tab)
    rad_acc = _k4(rd, rsw, rrow, rcnts)
    ang_acc = _k5(ta, td, ts, trow, tcnts)
    return _k6(ecfp, rad_acc, ang_acc)
